# Initial kernel scaffold; baseline (speedup 1.0000x reference)
import jax
import jax.numpy as jnp
from jax.experimental import pallas as pl


def kernel(x, edge_index, batch, params):
    raise NotImplementedError("write your pallas kernel here")


# speedup vs baseline: 16.1954x; 16.1954x over previous
import functools

import jax
import jax.numpy as jnp
from jax import lax
from jax.experimental import pallas as pl
from jax.experimental.pallas import tpu as pltpu
from jax.experimental.pallas import tpu_sc as plsc

N = 100000
E = 1600000
H = 64
G = 128

NP = 102400
NPR = NP // 128
EPR = 12544
EP = EPR * 128
PAD = EP - E
NTILE = 16
NWORK = 32
TSLICE = NP // NTILE
VROWS_T = EPR // NTILE
SROWS_W = EPR // NWORK
GRP = 8
NBIN = 144
PN_W = NP // NWORK
NEG = -3.0e38

_mesh = plsc.VectorSubcoreMesh(core_axis_name="c", subcore_axis_name="s")


def _wid():
    return lax.axis_index("c") * NTILE + lax.axis_index("s")


@functools.partial(
    pl.kernel,
    out_type=jax.ShapeDtypeStruct((2, NP), jnp.float32),
    mesh=_mesh,
    scratch_types=[
        pltpu.VMEM_SHARED((NP,), jnp.float32),
        pltpu.VMEM((GRP, 128), jnp.int32),
        pltpu.VMEM((128,), jnp.float32),
    ],
)
def _deg_kernel(dst_hbm, zeros_hbm, out_hbm, acc, dbuf, ones):
    c = lax.axis_index("c")
    s = lax.axis_index("s")
    w = c * NTILE + s
    for k in range(8):
        ones[pl.ds(k * 16, 16)] = jnp.ones((16,), jnp.float32)
    pltpu.sync_copy(zeros_hbm.at[pl.ds(s * TSLICE, TSLICE)],
                    acc.at[pl.ds(s * TSLICE, TSLICE)])
    plsc.subcore_barrier()

    def body(g, _):
        row0 = w * SROWS_W + g * GRP
        pltpu.sync_copy(dst_hbm.at[pl.ds(row0, GRP)], dbuf)
        for jj in range(GRP):
            pltpu.sync_copy(ones, acc.at[dbuf.at[jj]], add=True)
        return _

    lax.fori_loop(0, SROWS_W // GRP, body, None)
    plsc.subcore_barrier()
    pltpu.sync_copy(acc.at[pl.ds(s * TSLICE, TSLICE)],
                    out_hbm.at[c, pl.ds(s * TSLICE, TSLICE)])


@functools.partial(
    pl.kernel,
    out_type=jax.ShapeDtypeStruct((2, NP), jnp.float32),
    mesh=_mesh,
    scratch_types=[
        pltpu.VMEM_SHARED((NP,), jnp.float32),
        pltpu.VMEM((GRP, 128), jnp.int32),
        pltpu.VMEM((GRP, 128), jnp.int32),
        pltpu.VMEM((GRP, 128), jnp.float32),
        pltpu.SemaphoreType.DMA,
    ],
)
def _scalar_prop_kernel(src_hbm, dst_hbm, tab_hbm, zeros_hbm, out_hbm,
                        acc, sbuf, dbuf, vals, sem):
    c = lax.axis_index("c")
    s = lax.axis_index("s")
    w = c * NTILE + s
    pltpu.sync_copy(zeros_hbm.at[pl.ds(s * TSLICE, TSLICE)],
                    acc.at[pl.ds(s * TSLICE, TSLICE)])
    plsc.subcore_barrier()

    def body(g, _):
        row0 = w * SROWS_W + g * GRP
        pltpu.sync_copy(src_hbm.at[pl.ds(row0, GRP)], sbuf)
        pltpu.sync_copy(dst_hbm.at[pl.ds(row0, GRP)], dbuf)
        handles = []
        for jj in range(GRP):
            handles.append(
                pltpu.async_copy(tab_hbm.at[sbuf.at[jj]], vals.at[jj], sem))
        for h in handles:
            h.wait()
        for jj in range(GRP):
            pltpu.sync_copy(vals.at[jj], acc.at[dbuf.at[jj]], add=True)
        return _

    lax.fori_loop(0, SROWS_W // GRP, body, None)
    plsc.subcore_barrier()
    pltpu.sync_copy(acc.at[pl.ds(s * TSLICE, TSLICE)],
                    out_hbm.at[c, pl.ds(s * TSLICE, TSLICE)])


@functools.partial(
    pl.kernel,
    out_type=jax.ShapeDtypeStruct((4, NP, 16), jnp.float32),
    mesh=_mesh,
    scratch_types=[
        pltpu.VMEM_SHARED((NP, 16), jnp.float32),
        pltpu.VMEM((GRP, 128), jnp.int32),
        pltpu.VMEM((GRP, 128), jnp.int32),
        pltpu.VMEM((GRP, 128), jnp.int32),
        pltpu.VMEM((GRP, 128, 16), jnp.float32),
        pltpu.SemaphoreType.DMA,
    ],
    compiler_params=pltpu.CompilerParams(use_tc_tiling_on_sc=False),
)
def _vec_prop_kernel(src_hbm, dst_hbm, tab_hbm, zeros_hbm, out_hbm,
                     acc, sbuf, dbuf, sadj, rows, sem):
    c = lax.axis_index("c")
    s = lax.axis_index("s")
    for q in range(2):
        jq = 2 * c + q
        joff = jnp.full((16,), jq * NP, jnp.int32)
        pltpu.sync_copy(zeros_hbm.at[pl.ds(s * TSLICE, TSLICE)],
                        acc.at[pl.ds(s * TSLICE, TSLICE)])
        plsc.subcore_barrier()

        def body(g, _):
            row0 = s * VROWS_T + g * GRP
            pltpu.sync_copy(src_hbm.at[pl.ds(row0, GRP)], sbuf)
            pltpu.sync_copy(dst_hbm.at[pl.ds(row0, GRP)], dbuf)
            for jj in range(GRP):
                for k in range(8):
                    sl = pl.ds(k * 16, 16)
                    sadj[jj, sl] = sbuf[jj, sl] + joff
            handles = []
            for jj in range(GRP):
                handles.append(
                    pltpu.async_copy(tab_hbm.at[sadj.at[jj]], rows.at[jj], sem))
            for h in handles:
                h.wait()
            for jj in range(GRP):
                pltpu.sync_copy(rows.at[jj], acc.at[dbuf.at[jj]], add=True)
            return _

        lax.fori_loop(0, VROWS_T // GRP, body, None)
        plsc.subcore_barrier()
        pltpu.sync_copy(acc.at[pl.ds(s * TSLICE, TSLICE)],
                        out_hbm.at[jq, pl.ds(s * TSLICE, TSLICE)])


@functools.partial(
    pl.kernel,
    out_type=(
        jax.ShapeDtypeStruct((NWORK, G, H), jnp.float32),
        jax.ShapeDtypeStruct((NWORK, G, H), jnp.float32),
        jax.ShapeDtypeStruct((NWORK, G), jnp.float32),
    ),
    mesh=_mesh,
    scratch_types=[
        pltpu.VMEM((NBIN, H), jnp.float32),
        pltpu.VMEM((NBIN, H), jnp.float32),
        pltpu.VMEM((NBIN,), jnp.float32),
        pltpu.VMEM((4, 128, 16), jnp.float32),
        pltpu.VMEM((4, 128, 16), jnp.float32),
        pltpu.VMEM((144,), jnp.int32),
        pltpu.VMEM((144,), jnp.float32),
        pltpu.VMEM((4, 16), jnp.float32),
    ],
    compiler_params=pltpu.CompilerParams(needs_layout_passes=False,
                                         use_tc_tiling_on_sc=False),
)
def _pool_kernel(raw_hbm, v_hbm, dinv_hbm, batch_hbm, c3_hbm,
                 sum_hbm, max_hbm, cnt_hbm,
                 sumacc, maxacc, cntacc, rbuf, vbuf, bbuf, dvbuf, c3buf):
    c = lax.axis_index("c")
    s = lax.axis_index("s")
    w = c * NTILE + s
    pltpu.sync_copy(c3_hbm, c3buf)
    zero16 = jnp.zeros((16,), jnp.float32)
    neg16 = jnp.full((16,), NEG, jnp.float32)
    lane0 = jax.lax.iota(jnp.int32, 16) == 0
    one16 = jnp.ones((16,), jnp.float32)

    def initb(r, _):
        for k in range(4):
            sumacc[r, pl.ds(k * 16, 16)] = zero16
            maxacc[r, pl.ds(k * 16, 16)] = neg16
        return _

    lax.fori_loop(0, NBIN, initb, None)
    for k in range(NBIN // 16):
        cntacc[pl.ds(k * 16, 16)] = zero16

    cols = [jax.lax.iota(jnp.int32, 16) + 16 * q for q in range(4)]

    def chunk(ci, _):
        node0 = w * PN_W + ci * 128
        for q in range(4):
            pltpu.sync_copy(raw_hbm.at[q, pl.ds(node0, 128)], rbuf.at[q])
            pltpu.sync_copy(v_hbm.at[q, pl.ds(node0, 128)], vbuf.at[q])
        pltpu.sync_copy(batch_hbm.at[pl.ds(node0, 128)], bbuf.at[pl.ds(0, 128)])
        pltpu.sync_copy(dinv_hbm.at[pl.ds(node0, 128)], dvbuf.at[pl.ds(0, 128)])

        def node(i, _):
            gsc = bbuf[pl.ds(i, 16)][0]
            gv = jnp.full((16,), gsc, jnp.int32)
            dv = jnp.full((16,), dvbuf[pl.ds(i, 16)][0], jnp.float32)
            plsc.addupdate_scatter(cntacc, [gv], one16, mask=lane0)
            for q in range(4):
                hval = jnp.maximum(
                    dv * (rbuf[q, i] + vbuf[q, i]) + c3buf[q], 0.0)
                plsc.addupdate_scatter(sumacc, [gv, cols[q]], hval)
                cur = plsc.load_gather(maxacc, [gv, cols[q]])
                plsc.store_scatter(maxacc, [gv, cols[q]],
                                   jnp.maximum(cur, hval))
            return _

        lax.fori_loop(0, 128, node, None)
        return _

    lax.fori_loop(0, PN_W // 128, chunk, None)
    pltpu.sync_copy(sumacc.at[pl.ds(0, G)], sum_hbm.at[w])
    pltpu.sync_copy(maxacc.at[pl.ds(0, G)], max_hbm.at[w])
    pltpu.sync_copy(cntacc.at[pl.ds(0, G)], cnt_hbm.at[w])


def _deg_fin_body(degp_ref, x_ref, dinv_ref, xt_ref):
    deg = degp_ref[0] + degp_ref[1] + 1.0
    dinv = lax.rsqrt(jnp.maximum(deg, 1.0))
    dinv_ref[...] = dinv
    xt_ref[...] = x_ref[...] * dinv


def _deg_finalize(degp, x2d):
    return pl.pallas_call(
        _deg_fin_body,
        out_shape=(jax.ShapeDtypeStruct((NPR, 128), jnp.float32),
                   jax.ShapeDtypeStruct((NPR, 128), jnp.float32)),
    )(degp, x2d)


def _l1_body(pp_ref, dinv_ref, x_ref, a1_ref, c1_ref, w2_ref, v2_ref):
    ssum = pp_ref[0] + pp_ref[1]
    dinv = dinv_ref[...]
    p = dinv * ssum + dinv * dinv * x_ref[...]
    h1 = jnp.maximum(p * a1_ref[...] + c1_ref[...], 0.0)
    u2 = jnp.dot(h1, w2_ref[...], preferred_element_type=jnp.float32)
    v2 = dinv * u2
    v2_ref[...] = jnp.stack(
        [v2[:, 0:16], v2[:, 16:32], v2[:, 32:48], v2[:, 48:64]], axis=0)


def _l1_dense(pp, dinv, xp, a1, c1, w2g):
    BB = 1024
    nb = NP // BB
    return pl.pallas_call(
        _l1_body,
        grid=(nb,),
        in_specs=[
            pl.BlockSpec((2, BB, 1), lambda i: (0, i, 0)),
            pl.BlockSpec((BB, 1), lambda i: (i, 0)),
            pl.BlockSpec((BB, 1), lambda i: (i, 0)),
            pl.BlockSpec((1, H), lambda i: (0, 0)),
            pl.BlockSpec((1, H), lambda i: (0, 0)),
            pl.BlockSpec((H, H), lambda i: (0, 0)),
        ],
        out_specs=pl.BlockSpec((4, BB, 16), lambda i: (0, i, 0)),
        out_shape=jax.ShapeDtypeStruct((4, NP, 16), jnp.float32),
    )(pp, dinv, xp, a1, c1, w2g)


def _mid_body(raw_ref, v_ref, dinv_ref, cb_ref, w_ref, out_ref):
    dinv = dinv_ref[...]
    hq = []
    for q in range(4):
        hq.append(jnp.maximum(
            dinv * (raw_ref[q] + v_ref[q]) + cb_ref[q][None, :], 0.0))
    h = jnp.concatenate(hq, axis=1)
    u = jnp.dot(h, w_ref[...], preferred_element_type=jnp.float32)
    v = dinv * u
    out_ref[...] = jnp.stack(
        [v[:, 0:16], v[:, 16:32], v[:, 32:48], v[:, 48:64]], axis=0)


def _mid_dense(raw, vprev, dinv, cq, wg):
    BB = 1024
    nb = NP // BB
    return pl.pallas_call(
        _mid_body,
        grid=(nb,),
        in_specs=[
            pl.BlockSpec((4, BB, 16), lambda i: (0, i, 0)),
            pl.BlockSpec((4, BB, 16), lambda i: (0, i, 0)),
            pl.BlockSpec((BB, 1), lambda i: (i, 0)),
            pl.BlockSpec((4, 16), lambda i: (0, 0)),
            pl.BlockSpec((H, H), lambda i: (0, 0)),
        ],
        out_specs=pl.BlockSpec((4, BB, 16), lambda i: (0, i, 0)),
        out_shape=jax.ShapeDtypeStruct((4, NP, 16), jnp.float32),
    )(raw, vprev, dinv, cq, wg)


def _head_body(sp_ref, mp_ref, cp_ref, zw_ref, zc_ref, w2_ref, b2_ref,
               out_ref):
    sums = jnp.sum(sp_ref[...], axis=0)
    maxs = jnp.max(mp_ref[...], axis=0)
    cnt = jnp.sum(cp_ref[...], axis=0)
    mean = sums / jnp.maximum(cnt, 1.0)[:, None]
    xmax = jnp.where(maxs < 0.0, 0.0, maxs)
    z = jnp.concatenate([mean, xmax], axis=1)
    z = jnp.maximum(
        jnp.dot(z, zw_ref[...], preferred_element_type=jnp.float32)
        + zc_ref[...], 0.0)
    out_ref[...] = (jnp.dot(z, w2_ref[...], preferred_element_type=jnp.float32)
                    + b2_ref[...])


def _head(sp, mp, cp, zw, zc, wl2, bl2):
    return pl.pallas_call(
        _head_body,
        out_shape=jax.ShapeDtypeStruct((G, 2), jnp.float32),
    )(sp, mp, cp, zw, zc, wl2, bl2)


def kernel(x, edge_index, batch, params):
    f32 = jnp.float32
    src = edge_index[0]
    dst = edge_index[1]
    ar = jnp.arange(PAD, dtype=jnp.int32)
    src_p = jnp.concatenate([src, ar % N]).reshape(EPR, 128)
    dst_p = jnp.concatenate([dst, N + (ar % 64)]).reshape(EPR, 128)

    xr = x[:, 0]
    xp_flat = jnp.concatenate([xr, jnp.zeros((NP - N,), f32)])
    x2d = xp_flat.reshape(NPR, 128)
    batch_p = jnp.concatenate(
        [batch, jnp.full((NP - N,), G, batch.dtype)]).astype(jnp.int32)

    a1 = (params['W1'][0] * params['g1']).reshape(1, H)
    c1 = (params['b1'] * params['g1'] + params['be1']).reshape(1, H)
    w2g = params['W2'] * params['g2'][None, :]
    c2q = (params['b2'] * params['g2'] + params['be2']).reshape(4, 16)
    w3g = params['W3'] * params['g3'][None, :]
    c3q = (params['b3'] * params['g3'] + params['be3']).reshape(4, 16)
    zw = params['Wl1'] * params['gl'][None, :]
    zc = (params['bl1'] * params['gl'] + params['bel']).reshape(1, H)
    wl2 = params['Wl2']
    bl2 = params['bl2'].reshape(1, 2)

    zeros_n = jnp.zeros((NP,), f32)
    zeros_n16 = jnp.zeros((NP, 16), f32)

    degp = _deg_kernel(dst_p, zeros_n)
    dinv2d, xt2d = _deg_finalize(degp.reshape(2, NPR, 128), x2d)
    dinv_flat = dinv2d.reshape(NP)
    pp = _scalar_prop_kernel(src_p, dst_p, xt2d.reshape(NP), zeros_n)

    v2 = _l1_dense(pp.reshape(2, NP, 1), dinv_flat.reshape(NP, 1),
                   xp_flat.reshape(NP, 1), a1, c1, w2g)
    raw2 = _vec_prop_kernel(src_p, dst_p, v2.reshape(4 * NP, 16), zeros_n16)
    v3 = _mid_dense(raw2, v2, dinv_flat.reshape(NP, 1), c2q, w3g)
    raw3 = _vec_prop_kernel(src_p, dst_p, v3.reshape(4 * NP, 16), zeros_n16)

    sp, mp, cp = _pool_kernel(raw3, v3, dinv_flat, batch_p, c3q)
    return _head(sp, mp, cp, zw, zc, wl2, bl2)
